# Initial kernel scaffold; baseline (speedup 1.0000x reference)
import functools

import jax
import jax.numpy as jnp
from jax import lax
from jax.experimental import pallas as pl
from jax.experimental.pallas import tpu as pltpu
from jax.experimental.pallas import tpu_sc as plsc

N_NODES = 10000
DIM = 128
N_EDGES = 320000

K = 128
NC, NS = 2, 16
NW = NC * NS
NB = -(-N_EDGES // (K * NW))
NBLK = NB * NW
E_PAD = NBLK * K
ROWS_PAD = 10016
ZB = ROWS_PAD // NS
OB = N_NODES // NS
MMB = 1000


def _mm_kernel(x_ref, wt_ref, b_ref, o_ref):
    o_ref[...] = (
        jnp.dot(x_ref[...], wt_ref[...], preferred_element_type=jnp.float32,
                precision=jax.lax.Precision.HIGHEST)
        + b_ref[...]
    )


def _msg_matmul(x, wt, b):
    return pl.pallas_call(
        _mm_kernel,
        grid=(N_NODES // MMB,),
        in_specs=[
            pl.BlockSpec((MMB, DIM), lambda i: (i, 0)),
            pl.BlockSpec((DIM, DIM), lambda i: (0, 0)),
            pl.BlockSpec((1, DIM), lambda i: (0, 0)),
        ],
        out_specs=pl.BlockSpec((MMB, DIM), lambda i: (i, 0)),
        out_shape=jax.ShapeDtypeStruct((N_NODES, DIM), jnp.float32),
    )(x, wt, b.reshape(1, DIM))


def _final_kernel(s_ref, x_ref, p0_ref, p1_ref, wt_ref, b_ref, o_ref):
    t = x_ref[...] * s_ref[0, 0] + p0_ref[...] + p1_ref[...]
    y = (
        jnp.dot(t, wt_ref[...], preferred_element_type=jnp.float32,
                precision=jax.lax.Precision.HIGHEST)
        + b_ref[...]
    )
    o_ref[...] = jnp.maximum(y, 0.0)


def _final(s, x, parts, wt, b):
    return pl.pallas_call(
        _final_kernel,
        grid=(N_NODES // MMB,),
        in_specs=[
            pl.BlockSpec(memory_space=pltpu.SMEM),
            pl.BlockSpec((MMB, DIM), lambda i: (i, 0)),
            pl.BlockSpec((MMB, DIM), lambda i: (i, 0)),
            pl.BlockSpec((MMB, DIM), lambda i: (i + N_NODES // MMB, 0)),
            pl.BlockSpec((DIM, DIM), lambda i: (0, 0)),
            pl.BlockSpec((1, DIM), lambda i: (0, 0)),
        ],
        out_specs=pl.BlockSpec((MMB, DIM), lambda i: (i, 0)),
        out_shape=jax.ShapeDtypeStruct((N_NODES, DIM), jnp.float32),
    )(s, x, parts, parts, wt, b.reshape(1, DIM))


def _sc_scatter(msg, src_blk, dst_blk, zeros):
    mesh = plsc.VectorSubcoreMesh(core_axis_name="c", subcore_axis_name="s")

    @functools.partial(
        pl.kernel,
        out_type=jax.ShapeDtypeStruct((NC * N_NODES, DIM), jnp.float32),
        mesh=mesh,
        scratch_types=[
            pltpu.VMEM_SHARED((ROWS_PAD, DIM), jnp.float32),
            pltpu.VMEM((K,), jnp.int32),
            pltpu.VMEM((K,), jnp.int32),
            pltpu.VMEM((K,), jnp.int32),
            pltpu.VMEM((K,), jnp.int32),
            pltpu.VMEM((K, DIM), jnp.float32),
            pltpu.VMEM((K, DIM), jnp.float32),
            pltpu.SemaphoreType.DMA,
            pltpu.SemaphoreType.DMA,
        ],
    )
    def k(msg_hbm, src_hbm, dst_hbm, z_hbm, out_hbm,
          acc, di0, di1, si0, si1, r0, r1, g0, g1):
        cid = lax.axis_index("c")
        sid = lax.axis_index("s")

        pltpu.sync_copy(z_hbm.at[pl.ds(sid * ZB, ZB)],
                        acc.at[pl.ds(sid * ZB, ZB)])
        plsc.subcore_barrier()

        wid = cid * NS + sid
        b0 = wid * NB

        pltpu.sync_copy(dst_hbm.at[b0], di0)
        pltpu.sync_copy(src_hbm.at[b0], si0)
        pltpu.async_copy(msg_hbm.at[di0], r0, g0)

        @pl.loop(0, NB, step=2)
        def _(i):
            blk = b0 + i

            @pl.when(i + 1 < NB)
            def _():
                pltpu.sync_copy(dst_hbm.at[blk + 1], di1)
                pltpu.sync_copy(src_hbm.at[blk + 1], si1)
                pltpu.async_copy(msg_hbm.at[di1], r1, g1)

            pltpu.make_async_copy(msg_hbm.at[di0], r0, g0).wait()
            pltpu.sync_copy(r0, acc.at[si0], add=True)

            @pl.when(i + 2 < NB)
            def _():
                pltpu.sync_copy(dst_hbm.at[blk + 2], di0)
                pltpu.sync_copy(src_hbm.at[blk + 2], si0)
                pltpu.async_copy(msg_hbm.at[di0], r0, g0)

            @pl.when(i + 1 < NB)
            def _():
                pltpu.make_async_copy(msg_hbm.at[di1], r1, g1).wait()
                pltpu.sync_copy(r1, acc.at[si1], add=True)

        plsc.subcore_barrier()
        pltpu.sync_copy(acc.at[pl.ds(sid * OB, OB)],
                        out_hbm.at[pl.ds(cid * N_NODES + sid * OB, OB)])

    return k(msg, src_blk, dst_blk, zeros)


def kernel(X, edge_index, eps, W_msg, b_msg, W_lin, b_lin):
    src = edge_index[0]
    dst = edge_index[1]
    pad = E_PAD - N_EDGES
    fill = jnp.arange(pad, dtype=jnp.int32) % 16
    src_blk = jnp.concatenate([src, N_NODES + fill]).reshape(NBLK, K)
    dst_blk = jnp.concatenate([dst, fill]).reshape(NBLK, K)
    zeros = jnp.zeros((ROWS_PAD, DIM), jnp.float32)

    msg = _msg_matmul(X, W_msg.T, b_msg)
    parts = _sc_scatter(msg, src_blk, dst_blk, zeros)
    s = jnp.reshape(1.0 + eps, (1, 1)).astype(jnp.float32)
    return _final(s, X, parts, W_lin.T, b_lin)


# speedup vs baseline: 8.9335x; 8.9335x over previous
import functools

import jax
import jax.numpy as jnp
from jax import lax
from jax.experimental import pallas as pl
from jax.experimental.pallas import tpu as pltpu
from jax.experimental.pallas import tpu_sc as plsc

N_NODES = 10000
DIM = 128
N_EDGES = 320000

K = 128
NC, NS = 2, 16
NW = NC * NS
NB = -(-N_EDGES // (K * NW))
NBLK = NB * NW
E_PAD = NBLK * K
ROWS_PAD = 10112
ZB = ROWS_PAD // NS
MMB = 1000


def _mm_kernel(x_ref, wt_ref, b_ref, o_ref):
    o_ref[...] = (
        jnp.dot(x_ref[...], wt_ref[...], preferred_element_type=jnp.float32,
                precision=jax.lax.Precision.HIGHEST)
        + b_ref[...]
    )


def _msg_matmul(x, wt, b):
    return pl.pallas_call(
        _mm_kernel,
        grid=(N_NODES // MMB,),
        in_specs=[
            pl.BlockSpec((MMB, DIM), lambda i: (i, 0)),
            pl.BlockSpec((DIM, DIM), lambda i: (0, 0)),
            pl.BlockSpec((1, DIM), lambda i: (0, 0)),
        ],
        out_specs=pl.BlockSpec((MMB, DIM), lambda i: (i, 0)),
        out_shape=jax.ShapeDtypeStruct((N_NODES, DIM), jnp.float32),
    )(x, wt, b.reshape(1, DIM))


def _final_kernel(s_ref, x_ref, p0_ref, p1_ref, wt_ref, b_ref, o_ref):
    t = x_ref[...] * s_ref[0, 0] + p0_ref[...] + p1_ref[...]
    y = (
        jnp.dot(t, wt_ref[...], preferred_element_type=jnp.float32,
                precision=jax.lax.Precision.HIGHEST)
        + b_ref[...]
    )
    o_ref[...] = jnp.maximum(y, 0.0)


def _final(s, x, p0, p1, wt, b):
    return pl.pallas_call(
        _final_kernel,
        grid=(N_NODES // MMB,),
        in_specs=[
            pl.BlockSpec(memory_space=pltpu.SMEM),
            pl.BlockSpec((MMB, DIM), lambda i: (i, 0)),
            pl.BlockSpec((MMB, DIM), lambda i: (i, 0)),
            pl.BlockSpec((MMB, DIM), lambda i: (i, 0)),
            pl.BlockSpec((DIM, DIM), lambda i: (0, 0)),
            pl.BlockSpec((1, DIM), lambda i: (0, 0)),
        ],
        out_specs=pl.BlockSpec((MMB, DIM), lambda i: (i, 0)),
        out_shape=jax.ShapeDtypeStruct((N_NODES, DIM), jnp.float32),
    )(s, x, p0, p1, wt, b.reshape(1, DIM))


def _sc_scatter(msg, src_blk, dst_blk, zeros):
    mesh = plsc.VectorSubcoreMesh(core_axis_name="c", subcore_axis_name="s")

    @functools.partial(
        pl.kernel,
        out_type=jax.ShapeDtypeStruct((NC * ROWS_PAD, DIM), jnp.float32),
        mesh=mesh,
        scratch_types=[
            pltpu.VMEM_SHARED((ROWS_PAD, DIM), jnp.float32),
            pltpu.VMEM((1, K), jnp.int32),
            pltpu.VMEM((1, K), jnp.int32),
            pltpu.VMEM((1, K), jnp.int32),
            pltpu.VMEM((1, K), jnp.int32),
            pltpu.VMEM((K, DIM), jnp.float32),
            pltpu.VMEM((K, DIM), jnp.float32),
            pltpu.SemaphoreType.DMA,
            pltpu.SemaphoreType.DMA,
        ],
    )
    def k(msg_hbm, src_hbm, dst_hbm, z_hbm, out_hbm,
          acc, di0, di1, si0, si1, r0, r1, g0, g1):
        cid = lax.axis_index("c")
        sid = lax.axis_index("s")

        pltpu.sync_copy(z_hbm.at[pl.ds(sid * ZB, ZB)],
                        acc.at[pl.ds(sid * ZB, ZB)])
        plsc.subcore_barrier()

        wid = cid * NS + sid
        b0 = wid * NB

        pltpu.sync_copy(dst_hbm.at[b0], di0)
        pltpu.sync_copy(src_hbm.at[b0], si0)
        pltpu.async_copy(msg_hbm.at[di0.at[0]], r0, g0)

        @pl.loop(0, NB, step=2)
        def _(i):
            blk = b0 + i

            @pl.when(i + 1 < NB)
            def _():
                pltpu.sync_copy(dst_hbm.at[blk + 1], di1)
                pltpu.sync_copy(src_hbm.at[blk + 1], si1)
                pltpu.async_copy(msg_hbm.at[di1.at[0]], r1, g1)

            pltpu.make_async_copy(msg_hbm.at[di0.at[0]], r0, g0).wait()
            pltpu.sync_copy(r0, acc.at[si0.at[0]], add=True)

            @pl.when(i + 2 < NB)
            def _():
                pltpu.sync_copy(dst_hbm.at[blk + 2], di0)
                pltpu.sync_copy(src_hbm.at[blk + 2], si0)
                pltpu.async_copy(msg_hbm.at[di0.at[0]], r0, g0)

            @pl.when(i + 1 < NB)
            def _():
                pltpu.make_async_copy(msg_hbm.at[di1.at[0]], r1, g1).wait()
                pltpu.sync_copy(r1, acc.at[si1.at[0]], add=True)

        plsc.subcore_barrier()
        pltpu.sync_copy(acc.at[pl.ds(sid * ZB, ZB)],
                        out_hbm.at[pl.ds(cid * ROWS_PAD + sid * ZB, ZB)])

    return k(msg, src_blk, dst_blk, zeros)


def kernel(X, edge_index, eps, W_msg, b_msg, W_lin, b_lin):
    src = edge_index[0]
    dst = edge_index[1]
    pad = E_PAD - N_EDGES
    fill = jnp.arange(pad, dtype=jnp.int32) % 16
    src_blk = jnp.concatenate([src, N_NODES + fill]).reshape(NBLK, 1, K)
    dst_blk = jnp.concatenate([dst, fill]).reshape(NBLK, 1, K)
    zeros = jnp.zeros((ROWS_PAD, DIM), jnp.float32)

    msg = _msg_matmul(X, W_msg.T, b_msg)
    parts = _sc_scatter(msg, src_blk, dst_blk, zeros)
    p0 = lax.slice(parts, (0, 0), (N_NODES, DIM))
    p1 = lax.slice(parts, (ROWS_PAD, 0), (ROWS_PAD + N_NODES, DIM))
    s = jnp.reshape(1.0 + eps, (1, 1)).astype(jnp.float32)
    return _final(s, X, p0, p1, W_lin.T, b_lin)


# speedup vs baseline: 12.1241x; 1.3572x over previous
import functools

import jax
import jax.numpy as jnp
from jax import lax
from jax.experimental import pallas as pl
from jax.experimental.pallas import tpu as pltpu
from jax.experimental.pallas import tpu_sc as plsc

N_NODES = 10000
DIM = 128
N_EDGES = 320000

K = 128
NC, NS = 2, 16
NW = NC * NS
NB = 80
NBLK = NB * NW
E_PAD = NBLK * K
ROWS_PAD = 10240
ZB = ROWS_PAD // NS
MMB = 1000
GRID = N_NODES // MMB
EB = E_PAD // GRID


def _pro_kernel(x_ref, wt_ref, b_ref, s_ref, d_ref, o_ref, pk_ref, z_ref):
    o_ref[...] = (
        jnp.dot(x_ref[...], wt_ref[...], preferred_element_type=jnp.float32)
        + b_ref[...]
    )
    pk_ref[...] = (s_ref[...] << 16) | d_ref[...]
    z_ref[...] = jnp.zeros_like(z_ref)


def _prologue(x, wt, b, src_p, dst_p):
    return pl.pallas_call(
        _pro_kernel,
        grid=(GRID,),
        in_specs=[
            pl.BlockSpec((MMB, DIM), lambda i: (i, 0)),
            pl.BlockSpec((DIM, DIM), lambda i: (0, 0)),
            pl.BlockSpec((1, DIM), lambda i: (0, 0)),
            pl.BlockSpec((1, 1, EB), lambda i: (i, 0, 0)),
            pl.BlockSpec((1, 1, EB), lambda i: (i, 0, 0)),
        ],
        out_specs=[
            pl.BlockSpec((MMB, DIM), lambda i: (i, 0)),
            pl.BlockSpec((1, 1, EB), lambda i: (i, 0, 0)),
            pl.BlockSpec((ROWS_PAD // GRID, DIM), lambda i: (i, 0)),
        ],
        out_shape=[
            jax.ShapeDtypeStruct((N_NODES, DIM), jnp.float32),
            jax.ShapeDtypeStruct((GRID, 1, EB), jnp.int32),
            jax.ShapeDtypeStruct((ROWS_PAD, DIM), jnp.float32),
        ],
    )(x, wt, b.reshape(1, DIM), src_p.reshape(GRID, 1, EB),
      dst_p.reshape(GRID, 1, EB))


def _final_kernel(s_ref, x_ref, p0_ref, p1_ref, wt_ref, b_ref, o_ref):
    t = x_ref[...] * (s_ref[0, 0] + 1.0) + p0_ref[...] + p1_ref[...]
    y = (
        jnp.dot(t, wt_ref[...], preferred_element_type=jnp.float32)
        + b_ref[...]
    )
    o_ref[...] = jnp.maximum(y, 0.0)


def _final(s, x, p0, p1, wt, b):
    return pl.pallas_call(
        _final_kernel,
        grid=(GRID,),
        in_specs=[
            pl.BlockSpec(memory_space=pltpu.SMEM),
            pl.BlockSpec((MMB, DIM), lambda i: (i, 0)),
            pl.BlockSpec((MMB, DIM), lambda i: (i, 0)),
            pl.BlockSpec((MMB, DIM), lambda i: (i, 0)),
            pl.BlockSpec((DIM, DIM), lambda i: (0, 0)),
            pl.BlockSpec((1, DIM), lambda i: (0, 0)),
        ],
        out_specs=pl.BlockSpec((MMB, DIM), lambda i: (i, 0)),
        out_shape=jax.ShapeDtypeStruct((N_NODES, DIM), jnp.float32),
    )(s, x, p0, p1, wt, b.reshape(1, DIM))


def _sc_scatter(msg, pk, zeros):
    mesh = plsc.VectorSubcoreMesh(core_axis_name="c", subcore_axis_name="s")

    @functools.partial(
        pl.kernel,
        out_type=(jax.ShapeDtypeStruct((ROWS_PAD, DIM), jnp.float32),
                  jax.ShapeDtypeStruct((ROWS_PAD, DIM), jnp.float32)),
        mesh=mesh,
        scratch_types=[
            pltpu.VMEM_SHARED((ROWS_PAD, DIM), jnp.float32),
            pltpu.VMEM((NB, K), jnp.int32),
            pltpu.VMEM((NB, K), jnp.int32),
            pltpu.VMEM((K, DIM), jnp.float32),
            pltpu.VMEM((K, DIM), jnp.float32),
            pltpu.SemaphoreType.DMA,
            pltpu.SemaphoreType.DMA,
        ],
    )
    def k(msg_hbm, src_hbm, dst_hbm, z_hbm, out_hbm,
          acc, di_all, si_all, r0, r1, g0, g1):
        cid = lax.axis_index("c")
        sid = lax.axis_index("s")

        pltpu.sync_copy(z_hbm.at[pl.ds(sid * ZB, ZB)],
                        acc.at[pl.ds(sid * ZB, ZB)])
        plsc.subcore_barrier()

        wid = cid * NS + sid

        pltpu.sync_copy(dst_hbm.at[wid], di_all)
        pltpu.sync_copy(src_hbm.at[wid], si_all)
        pltpu.async_copy(msg_hbm.at[di_all.at[0]], r0, g0)

        @pl.loop(0, NB, step=2)
        def _(i):
            @pl.when(i + 1 < NB)
            def _():
                pltpu.async_copy(msg_hbm.at[di_all.at[i + 1]], r1, g1)

            pltpu.make_async_copy(msg_hbm.at[di_all.at[i]], r0, g0).wait()
            pltpu.sync_copy(r0, acc.at[si_all.at[i]], add=True)

            @pl.when(i + 2 < NB)
            def _():
                pltpu.async_copy(msg_hbm.at[di_all.at[i + 2]], r0, g0)

            @pl.when(i + 1 < NB)
            def _():
                pltpu.make_async_copy(msg_hbm.at[di_all.at[i + 1]], r1, g1).wait()
                pltpu.sync_copy(r1, acc.at[si_all.at[i + 1]], add=True)

        plsc.subcore_barrier()

        @pl.when(cid == 0)
        def _():
            pltpu.sync_copy(acc.at[pl.ds(sid * ZB, ZB)],
                            out0_hbm.at[pl.ds(sid * ZB, ZB)])

        @pl.when(cid == 1)
        def _():
            pltpu.sync_copy(acc.at[pl.ds(sid * ZB, ZB)],
                            out1_hbm.at[pl.ds(sid * ZB, ZB)])

    return k(msg, src_blk, dst_blk, zeros)


def kernel(X, edge_index, eps, W_msg, b_msg, W_lin, b_lin):
    src = edge_index[0]
    dst = edge_index[1]
    pad = E_PAD - N_EDGES
    ar = jnp.arange(pad, dtype=jnp.int32)
    src_p = jnp.concatenate([src, N_NODES + ar % (ROWS_PAD - N_NODES)])
    dst_p = jnp.concatenate([dst, ar % 8192])

    msg, pk, zeros = _prologue(X, W_msg.T, b_msg, src_p, dst_p)
    p0, p1 = _sc_scatter(msg, pk.reshape(NW, NB, K), zeros)
    s = jnp.reshape(eps, (1, 1)).astype(jnp.float32)
    return _final(s, X, p0, p1, W_lin.T, b_lin)


# speedup vs baseline: 13.3425x; 1.1005x over previous
import functools

import jax
import jax.numpy as jnp
from jax import lax
from jax.experimental import pallas as pl
from jax.experimental.pallas import tpu as pltpu
from jax.experimental.pallas import tpu_sc as plsc

N_NODES = 10000
DIM = 128
N_EDGES = 320000

K = 128
NC, NS = 2, 16
NW = NC * NS
NB = 80
NBLK = NB * NW
E_PAD = NBLK * K
ROWS_PAD = 10112
ZB = ROWS_PAD // NS
MMB = 1000


def _mm_kernel(x_ref, wt_ref, b_ref, o_ref):
    o_ref[...] = (
        jnp.dot(x_ref[...], wt_ref[...], preferred_element_type=jnp.float32)
        + b_ref[...]
    )


def _msg_matmul(x, wt, b):
    return pl.pallas_call(
        _mm_kernel,
        grid=(N_NODES // MMB,),
        in_specs=[
            pl.BlockSpec((MMB, DIM), lambda i: (i, 0)),
            pl.BlockSpec((DIM, DIM), lambda i: (0, 0)),
            pl.BlockSpec((1, DIM), lambda i: (0, 0)),
        ],
        out_specs=pl.BlockSpec((MMB, DIM), lambda i: (i, 0)),
        out_shape=jax.ShapeDtypeStruct((N_NODES, DIM), jnp.float32),
    )(x, wt, b.reshape(1, DIM))


def _final_kernel(s_ref, x_ref, p0_ref, p1_ref, wt_ref, b_ref, o_ref):
    t = x_ref[...] * s_ref[0, 0] + p0_ref[...] + p1_ref[...]
    y = (
        jnp.dot(t, wt_ref[...], preferred_element_type=jnp.float32)
        + b_ref[...]
    )
    o_ref[...] = jnp.maximum(y, 0.0)


def _final(s, x, p0, p1, wt, b):
    return pl.pallas_call(
        _final_kernel,
        grid=(N_NODES // MMB,),
        in_specs=[
            pl.BlockSpec(memory_space=pltpu.SMEM),
            pl.BlockSpec((MMB, DIM), lambda i: (i, 0)),
            pl.BlockSpec((MMB, DIM), lambda i: (i, 0)),
            pl.BlockSpec((MMB, DIM), lambda i: (i, 0)),
            pl.BlockSpec((DIM, DIM), lambda i: (0, 0)),
            pl.BlockSpec((1, DIM), lambda i: (0, 0)),
        ],
        out_specs=pl.BlockSpec((MMB, DIM), lambda i: (i, 0)),
        out_shape=jax.ShapeDtypeStruct((N_NODES, DIM), jnp.float32),
    )(s, x, p0, p1, wt, b.reshape(1, DIM))


def _sc_scatter(msg, pk, zeros):
    mesh = plsc.VectorSubcoreMesh(core_axis_name="c", subcore_axis_name="s")

    @functools.partial(
        pl.kernel,
        out_type=(jax.ShapeDtypeStruct((ROWS_PAD, DIM), jnp.float32),
                  jax.ShapeDtypeStruct((ROWS_PAD, DIM), jnp.float32)),
        mesh=mesh,
        scratch_types=[
            pltpu.VMEM_SHARED((ROWS_PAD, DIM), jnp.float32),
            pltpu.VMEM((NB, K), jnp.int32),
            pltpu.VMEM((NB, K), jnp.int32),
            pltpu.VMEM((K, DIM), jnp.float32),
            pltpu.VMEM((K, DIM), jnp.float32),
            pltpu.SemaphoreType.DMA,
            pltpu.SemaphoreType.DMA,
        ],
    )
    def k(msg_hbm, src_hbm, dst_hbm, z_hbm, out_hbm,
          acc, di_all, si_all, r0, r1, g0, g1):
        cid = lax.axis_index("c")
        sid = lax.axis_index("s")

        pltpu.sync_copy(z_hbm.at[pl.ds(sid * ZB, ZB)],
                        acc.at[pl.ds(sid * ZB, ZB)])
        plsc.subcore_barrier()

        wid = cid * NS + sid

        pltpu.sync_copy(dst_hbm.at[wid], di_all)
        pltpu.sync_copy(src_hbm.at[wid], si_all)
        pltpu.async_copy(msg_hbm.at[di_all.at[0]], r0, g0)

        @pl.loop(0, NB, step=2)
        def _(i):
            @pl.when(i + 1 < NB)
            def _():
                pltpu.async_copy(msg_hbm.at[di_all.at[i + 1]], r1, g1)

            pltpu.make_async_copy(msg_hbm.at[di_all.at[i]], r0, g0).wait()
            pltpu.sync_copy(r0, acc.at[si_all.at[i]], add=True)

            @pl.when(i + 2 < NB)
            def _():
                pltpu.async_copy(msg_hbm.at[di_all.at[i + 2]], r0, g0)

            @pl.when(i + 1 < NB)
            def _():
                pltpu.make_async_copy(msg_hbm.at[di_all.at[i + 1]], r1, g1).wait()
                pltpu.sync_copy(r1, acc.at[si_all.at[i + 1]], add=True)

        plsc.subcore_barrier()

        @pl.when(cid == 0)
        def _():
            pltpu.sync_copy(acc.at[pl.ds(sid * ZB, ZB)],
                            out0_hbm.at[pl.ds(sid * ZB, ZB)])

        @pl.when(cid == 1)
        def _():
            pltpu.sync_copy(acc.at[pl.ds(sid * ZB, ZB)],
                            out1_hbm.at[pl.ds(sid * ZB, ZB)])

    return k(msg, src_blk, dst_blk, zeros)


def kernel(X, edge_index, eps, W_msg, b_msg, W_lin, b_lin):
    src = edge_index[0]
    dst = edge_index[1]
    pad = E_PAD - N_EDGES
    ar = jnp.arange(pad, dtype=jnp.int32)
    src_p = jnp.concatenate([src, N_NODES + ar % (ROWS_PAD - N_NODES)])
    dst_p = jnp.concatenate([dst, ar % 8192])
    pk = ((src_p << 16) | dst_p).reshape(NW, NB, K)
    zeros = jnp.zeros((ROWS_PAD, DIM), jnp.float32)

    msg = _msg_matmul(X, W_msg.T, b_msg)
    p0, p1 = _sc_scatter(msg, pk, zeros)
    s = jnp.reshape(1.0 + eps, (1, 1)).astype(jnp.float32)
    return _final(s, X, p0, p1, W_lin.T, b_lin)
